# baseline (speedup 1.0000x reference)
import functools

import jax
import jax.numpy as jnp
from jax import lax
from jax.experimental import pallas as pl
from jax.experimental.pallas import tpu as pltpu
from jax.experimental.pallas import tpu_sc as plsc

_NC = 2
_NS = 16
_NW = _NC * _NS
_LANES = 16


def _sc_scores(center, pos, neg_flat, w_in, w_out, *, B, K, D):
    BPW = B // _NW
    CHUNK = 32
    NCHUNK = BPW // CHUNK
    GROUPS = CHUNK // _LANES
    NPC = CHUNK * K
    IDXV = 128
    NEG_GATHERS = NPC // IDXV

    mesh = plsc.VectorSubcoreMesh(core_axis_name="c", subcore_axis_name="s")

    @functools.partial(
        pl.kernel,
        mesh=mesh,
        compiler_params=pltpu.CompilerParams(
            needs_layout_passes=False, use_tc_tiling_on_sc=False),
        out_type=[
            jax.ShapeDtypeStruct((B,), jnp.float32),
            jax.ShapeDtypeStruct((B * K,), jnp.float32),
        ],
        scratch_types=[
            pltpu.VMEM((BPW,), jnp.int32),
            pltpu.VMEM((BPW,), jnp.int32),
            pltpu.VMEM((BPW * K,), jnp.int32),
            pltpu.VMEM((CHUNK, D), jnp.float32),
            pltpu.VMEM((CHUNK, D), jnp.float32),
            pltpu.VMEM((CHUNK, D), jnp.float32),
            pltpu.VMEM((CHUNK, D), jnp.float32),
            pltpu.VMEM((NPC, D), jnp.float32),
            pltpu.VMEM((NPC, D), jnp.float32),
            pltpu.VMEM((BPW,), jnp.float32),
            pltpu.VMEM((BPW * K,), jnp.float32),
            pltpu.SemaphoreType.DMA,
            pltpu.SemaphoreType.DMA,
        ],
    )
    def sc_kernel(center_h, pos_h, neg_h, win_h, wout_h, pos_out, neg_out,
                  cen_idx, pos_idx, neg_idx, cen0, cen1, posb0, posb1,
                  negb0, negb1, pos_sc, neg_sc, sem0, sem1):
        wid = lax.axis_index("s") * _NC + lax.axis_index("c")
        ebase = wid * BPW
        pltpu.sync_copy(center_h.at[pl.ds(ebase, BPW)], cen_idx)
        pltpu.sync_copy(pos_h.at[pl.ds(ebase, BPW)], pos_idx)
        pltpu.sync_copy(neg_h.at[pl.ds(ebase * K, BPW * K)], neg_idx)

        cenb = (cen0, cen1)
        posb = (posb0, posb1)
        negb = (negb0, negb1)
        sems = (sem0, sem1)

        def issue(c, slot):
            pltpu.async_copy(
                win_h.at[cen_idx.at[pl.ds(c * CHUNK, CHUNK)]],
                cenb[slot], sems[slot])
            pltpu.async_copy(
                wout_h.at[pos_idx.at[pl.ds(c * CHUNK, CHUNK)]],
                posb[slot], sems[slot])
            for j in range(NEG_GATHERS):
                pltpu.async_copy(
                    wout_h.at[neg_idx.at[pl.ds(c * NPC + j * IDXV, IDXV)]],
                    negb[slot].at[pl.ds(j * IDXV, IDXV)],
                    sems[slot])

        def drain(slot):
            pltpu.make_async_copy(
                win_h.at[pl.ds(0, CHUNK)],
                cenb[slot], sems[slot]).wait()
            pltpu.make_async_copy(
                wout_h.at[pl.ds(0, CHUNK)],
                posb[slot], sems[slot]).wait()
            pltpu.make_async_copy(
                wout_h.at[pl.ds(0, NPC)],
                negb[slot], sems[slot]).wait()

        lanes = lax.iota(jnp.int32, _LANES)

        def compute(c, slot):
            for g in range(GROUPS):
                rows = g * _LANES + lanes
                nrows = rows * K

                init = (jnp.zeros((_LANES,), jnp.float32),) * (K + 1)

                @plsc.parallel_loop(0, D, carry=init, unroll=2)
                def accs(d, accs):
                    dv = jnp.full((_LANES,), d, jnp.int32)
                    m = plsc.load_gather(cenb[slot], [rows, dv])
                    p = plsc.load_gather(posb[slot], [rows, dv])
                    out = [accs[0] + m * p]
                    for k in range(K):
                        nk = plsc.load_gather(negb[slot], [nrows + k, dv])
                        out.append(accs[1 + k] + m * nk)
                    return tuple(out)
                sbase = c * CHUNK + g * _LANES
                pos_sc[pl.ds(sbase, _LANES)] = accs[0]
                nbase = c * NPC + g * _LANES * K
                for k in range(K):
                    neg_sc[pl.ds(nbase + k * _LANES, _LANES)] = accs[1 + k]

        issue(0, 0)

        def pair_body(i, carry):
            for b in range(2):
                c = i * 2 + b
                nxt = c + 1

                @pl.when(nxt < NCHUNK)
                def _():
                    issue(nxt, (b + 1) % 2)

                drain(b)
                compute(c, b)
            return carry

        lax.fori_loop(0, NCHUNK // 2, pair_body, 0)

        pltpu.sync_copy(pos_sc, pos_out.at[pl.ds(ebase, BPW)])
        pltpu.sync_copy(neg_sc, neg_out.at[pl.ds(ebase * K, BPW * K)])

    return sc_kernel(center, pos, neg_flat, w_in, w_out)


def _tc_loss(pos_s, neg_s, B):
    def body(p_ref, n_ref, o_ref):
        p = p_ref[...]
        n = n_ref[...]

        def softplus(x):
            return jnp.maximum(x, 0.0) + jnp.log(1.0 + jnp.exp(-jnp.abs(x)))

        o_ref[0, 0] = (jnp.sum(softplus(-p)) + jnp.sum(softplus(n))) / B

    return pl.pallas_call(
        body,
        out_shape=jax.ShapeDtypeStruct((1, 1), jnp.float32),
        out_specs=pl.BlockSpec(memory_space=pltpu.SMEM),
    )(pos_s, neg_s)


def kernel(center, pos, neg, W_in, W_out):
    V, D = W_in.shape
    B, K = neg.shape
    center = center.astype(jnp.int32)
    pos = pos.astype(jnp.int32)
    neg_flat = neg.astype(jnp.int32).reshape(B * K)
    pos_s, neg_s = _sc_scores(center, pos, neg_flat, W_in, W_out, B=B, K=K, D=D)
    loss = _tc_loss(pos_s.reshape(B // 128, 128),
                    neg_s.reshape(B * K // 128, 128), B)
    return loss.reshape(())


# speedup vs baseline: 1.2864x; 1.2864x over previous
import functools

import jax
import jax.numpy as jnp
from jax import lax
from jax.experimental import pallas as pl
from jax.experimental.pallas import tpu as pltpu
from jax.experimental.pallas import tpu_sc as plsc

_NC = 2
_NS = 16
_NW = _NC * _NS
_LANES = 16


def _sc_scores(center, pos, neg_flat, w_in, w_out, *, B, K, D):
    BPW = B // _NW
    CHUNK = 32
    NCHUNK = BPW // CHUNK
    GROUPS = CHUNK // _LANES
    NPC = CHUNK * K
    IDXV = 128
    NEG_GATHERS = NPC // IDXV

    mesh = plsc.VectorSubcoreMesh(core_axis_name="c", subcore_axis_name="s")

    @functools.partial(
        pl.kernel,
        mesh=mesh,
        compiler_params=pltpu.CompilerParams(
            needs_layout_passes=False, use_tc_tiling_on_sc=False),
        out_type=[
            jax.ShapeDtypeStruct((B,), jnp.float32),
            jax.ShapeDtypeStruct((B * K,), jnp.float32),
        ],
        scratch_types=[
            pltpu.VMEM((BPW,), jnp.int32),
            pltpu.VMEM((BPW,), jnp.int32),
            pltpu.VMEM((BPW * K,), jnp.int32),
            pltpu.VMEM((CHUNK, D), jnp.float32),
            pltpu.VMEM((CHUNK, D), jnp.float32),
            pltpu.VMEM((CHUNK, D), jnp.float32),
            pltpu.VMEM((CHUNK, D), jnp.float32),
            pltpu.VMEM((NPC, D), jnp.float32),
            pltpu.VMEM((NPC, D), jnp.float32),
            pltpu.VMEM((BPW,), jnp.float32),
            pltpu.VMEM((BPW * K,), jnp.float32),
            pltpu.SemaphoreType.DMA,
            pltpu.SemaphoreType.DMA,
        ],
    )
    def sc_kernel(center_h, pos_h, neg_h, win_h, wout_h, pos_out, neg_out,
                  cen_idx, pos_idx, neg_idx, cen0, cen1, posb0, posb1,
                  negb0, negb1, pos_sc, neg_sc, sem0, sem1):
        wid = lax.axis_index("s") * _NC + lax.axis_index("c")
        ebase = wid * BPW
        pltpu.sync_copy(center_h.at[pl.ds(ebase, BPW)], cen_idx)
        pltpu.sync_copy(pos_h.at[pl.ds(ebase, BPW)], pos_idx)
        pltpu.sync_copy(neg_h.at[pl.ds(ebase * K, BPW * K)], neg_idx)

        cenb = (cen0, cen1)
        posb = (posb0, posb1)
        negb = (negb0, negb1)
        sems = (sem0, sem1)

        def issue(c, slot):
            pltpu.async_copy(
                win_h.at[cen_idx.at[pl.ds(c * CHUNK, CHUNK)]],
                cenb[slot], sems[slot])
            pltpu.async_copy(
                wout_h.at[pos_idx.at[pl.ds(c * CHUNK, CHUNK)]],
                posb[slot], sems[slot])
            for j in range(NEG_GATHERS):
                pltpu.async_copy(
                    wout_h.at[neg_idx.at[pl.ds(c * NPC + j * IDXV, IDXV)]],
                    negb[slot].at[pl.ds(j * IDXV, IDXV)],
                    sems[slot])

        def drain(slot):
            pltpu.make_async_copy(
                win_h.at[pl.ds(0, CHUNK)],
                cenb[slot], sems[slot]).wait()
            pltpu.make_async_copy(
                wout_h.at[pl.ds(0, CHUNK)],
                posb[slot], sems[slot]).wait()
            pltpu.make_async_copy(
                wout_h.at[pl.ds(0, NPC)],
                negb[slot], sems[slot]).wait()

        NJ = D // _LANES
        lanes = lax.iota(jnp.int32, _LANES)
        mask0 = lanes == 0

        def lane_sum(v):
            for s in (8, 4, 2, 1):
                v = v + jnp.take_along_axis(v, lanes ^ s, axis=0)
            return v

        def compute(c, slot):
            @plsc.parallel_loop(0, CHUNK)
            def _(e):
                m = [cenb[slot][e, pl.ds(_LANES * j, _LANES)]
                     for j in range(NJ)]
                p = [posb[slot][e, pl.ds(_LANES * j, _LANES)]
                     for j in range(NJ)]
                ps = m[0] * p[0]
                for j in range(1, NJ):
                    ps = ps + m[j] * p[j]
                t = jnp.full((_LANES,), c * CHUNK + e, jnp.int32)
                plsc.store_scatter(pos_sc, [t], lane_sum(ps), mask=mask0)
                for k in range(K):
                    n = [negb[slot][e * K + k, pl.ds(_LANES * j, _LANES)]
                         for j in range(NJ)]
                    ns = m[0] * n[0]
                    for j in range(1, NJ):
                        ns = ns + m[j] * n[j]
                    tn = jnp.full((_LANES,), (c * CHUNK + e) * K + k,
                                  jnp.int32)
                    plsc.store_scatter(neg_sc, [tn], lane_sum(ns), mask=mask0)

        issue(0, 0)

        def pair_body(i, carry):
            for b in range(2):
                c = i * 2 + b
                nxt = c + 1

                @pl.when(nxt < NCHUNK)
                def _():
                    issue(nxt, (b + 1) % 2)

                drain(b)
                compute(c, b)
            return carry

        lax.fori_loop(0, NCHUNK // 2, pair_body, 0)

        pltpu.sync_copy(pos_sc, pos_out.at[pl.ds(ebase, BPW)])
        pltpu.sync_copy(neg_sc, neg_out.at[pl.ds(ebase * K, BPW * K)])

    return sc_kernel(center, pos, neg_flat, w_in, w_out)


def _tc_loss(pos_s, neg_s, B):
    def body(p_ref, n_ref, o_ref):
        p = p_ref[...]
        n = n_ref[...]

        def softplus(x):
            return jnp.maximum(x, 0.0) + jnp.log(1.0 + jnp.exp(-jnp.abs(x)))

        o_ref[0, 0] = (jnp.sum(softplus(-p)) + jnp.sum(softplus(n))) / B

    return pl.pallas_call(
        body,
        out_shape=jax.ShapeDtypeStruct((1, 1), jnp.float32),
        out_specs=pl.BlockSpec(memory_space=pltpu.SMEM),
    )(pos_s, neg_s)


def kernel(center, pos, neg, W_in, W_out):
    V, D = W_in.shape
    B, K = neg.shape
    center = center.astype(jnp.int32)
    pos = pos.astype(jnp.int32)
    neg_flat = neg.astype(jnp.int32).reshape(B * K)
    pos_s, neg_s = _sc_scores(center, pos, neg_flat, W_in, W_out, B=B, K=K, D=D)
    loss = _tc_loss(pos_s.reshape(B // 128, 128),
                    neg_s.reshape(B * K // 128, 128), B)
    return loss.reshape(())


# speedup vs baseline: 1.3337x; 1.0368x over previous
import functools

import jax
import jax.numpy as jnp
from jax import lax
from jax.experimental import pallas as pl
from jax.experimental.pallas import tpu as pltpu
from jax.experimental.pallas import tpu_sc as plsc

_NC = 2
_NS = 16
_NW = _NC * _NS
_LANES = 16


def _sc_scores(center, pos, neg_flat, w_in, w_out, *, B, K, D):
    BPW = B // _NW
    W = 2 * D
    CHUNK = 16
    NCHUNK = BPW // CHUNK
    GROUPS = CHUNK // _LANES
    NPC = CHUNK * K
    IDXV = 64
    NEG_GATHERS = NPC // IDXV

    mesh = plsc.VectorSubcoreMesh(core_axis_name="c", subcore_axis_name="s")

    @functools.partial(
        pl.kernel,
        mesh=mesh,
        compiler_params=pltpu.CompilerParams(
            needs_layout_passes=False, use_tc_tiling_on_sc=True),
        out_type=[
            jax.ShapeDtypeStruct((B,), jnp.float32),
            jax.ShapeDtypeStruct((B * K,), jnp.float32),
        ],
        scratch_types=[
            pltpu.VMEM((BPW,), jnp.int32),
            pltpu.VMEM((BPW,), jnp.int32),
            pltpu.VMEM((BPW * K,), jnp.int32),
            pltpu.VMEM((CHUNK, W), jnp.float32),
            pltpu.VMEM((CHUNK, W), jnp.float32),
            pltpu.VMEM((CHUNK, W), jnp.float32),
            pltpu.VMEM((CHUNK, W), jnp.float32),
            pltpu.VMEM((NPC, W), jnp.float32),
            pltpu.VMEM((NPC, W), jnp.float32),
            pltpu.VMEM((BPW,), jnp.float32),
            pltpu.VMEM((BPW * K,), jnp.float32),
            pltpu.SemaphoreType.DMA,
            pltpu.SemaphoreType.DMA,
        ],
    )
    def sc_kernel(center_h, pos_h, neg_h, win_h, wout_h, pos_out, neg_out,
                  cen_idx, pos_idx, neg_idx, cen0, cen1, posb0, posb1,
                  negb0, negb1, pos_sc, neg_sc, sem0, sem1):
        wid = lax.axis_index("s") * _NC + lax.axis_index("c")
        ebase = wid * BPW
        pltpu.sync_copy(center_h.at[pl.ds(ebase, BPW)], cen_idx)
        pltpu.sync_copy(pos_h.at[pl.ds(ebase, BPW)], pos_idx)
        pltpu.sync_copy(neg_h.at[pl.ds(ebase * K, BPW * K)], neg_idx)

        cenb = (cen0, cen1)
        posb = (posb0, posb1)
        negb = (negb0, negb1)
        sems = (sem0, sem1)

        def issue(c, slot):
            pltpu.async_copy(
                win_h.at[cen_idx.at[pl.ds(c * CHUNK, CHUNK)]],
                cenb[slot], sems[slot])
            pltpu.async_copy(
                wout_h.at[pos_idx.at[pl.ds(c * CHUNK, CHUNK)]],
                posb[slot], sems[slot])
            for j in range(NEG_GATHERS):
                pltpu.async_copy(
                    wout_h.at[neg_idx.at[pl.ds(c * NPC + j * IDXV, IDXV)]],
                    negb[slot].at[pl.ds(j * IDXV, IDXV)],
                    sems[slot])

        def drain(slot):
            pltpu.make_async_copy(
                win_h.at[pl.ds(0, CHUNK)],
                cenb[slot], sems[slot]).wait()
            pltpu.make_async_copy(
                wout_h.at[pl.ds(0, CHUNK)],
                posb[slot], sems[slot]).wait()
            pltpu.make_async_copy(
                wout_h.at[pl.ds(0, NPC)],
                negb[slot], sems[slot]).wait()

        NJ = D // _LANES
        lanes = lax.iota(jnp.int32, _LANES)
        mask0 = lanes == 0

        def lane_sum(v):
            for s in (8, 4, 2, 1):
                v = v + jnp.take_along_axis(v, lanes ^ s, axis=0)
            return v

        def compute(c, slot):
            @plsc.parallel_loop(0, CHUNK)
            def _(e):
                m = [cenb[slot][e, pl.ds(_LANES * j, _LANES)]
                     for j in range(NJ)]
                p = [posb[slot][e, pl.ds(_LANES * j, _LANES)]
                     for j in range(NJ)]
                ps = m[0] * p[0]
                for j in range(1, NJ):
                    ps = ps + m[j] * p[j]
                t = jnp.full((_LANES,), c * CHUNK + e, jnp.int32)
                plsc.store_scatter(pos_sc, [t], lane_sum(ps), mask=mask0)
                for k in range(K):
                    n = [negb[slot][e * K + k, pl.ds(_LANES * j, _LANES)]
                         for j in range(NJ)]
                    ns = m[0] * n[0]
                    for j in range(1, NJ):
                        ns = ns + m[j] * n[j]
                    tn = jnp.full((_LANES,), (c * CHUNK + e) * K + k,
                                  jnp.int32)
                    plsc.store_scatter(neg_sc, [tn], lane_sum(ns), mask=mask0)

        issue(0, 0)

        def pair_body(i, carry):
            for b in range(2):
                c = i * 2 + b
                nxt = c + 1

                @pl.when(nxt < NCHUNK)
                def _():
                    issue(nxt, (b + 1) % 2)

                drain(b)
                compute(c, b)
            return carry

        lax.fori_loop(0, NCHUNK // 2, pair_body, 0)

        pltpu.sync_copy(pos_sc, pos_out.at[pl.ds(ebase, BPW)])
        pltpu.sync_copy(neg_sc, neg_out.at[pl.ds(ebase * K, BPW * K)])

    return sc_kernel(center, pos, neg_flat, w_in, w_out)


def _tc_loss(pos_s, neg_s, B):
    def body(p_ref, n_ref, o_ref):
        p = p_ref[...]
        n = n_ref[...]

        def softplus(x):
            return jnp.maximum(x, 0.0) + jnp.log(1.0 + jnp.exp(-jnp.abs(x)))

        o_ref[0, 0] = (jnp.sum(softplus(-p)) + jnp.sum(softplus(n))) / B

    return pl.pallas_call(
        body,
        out_shape=jax.ShapeDtypeStruct((1, 1), jnp.float32),
        out_specs=pl.BlockSpec(memory_space=pltpu.SMEM),
    )(pos_s, neg_s)


def kernel(center, pos, neg, W_in, W_out):
    V, D = W_in.shape
    B, K = neg.shape
    center = center.astype(jnp.int32)
    pos = pos.astype(jnp.int32)
    neg_flat = neg.astype(jnp.int32).reshape(B * K)
    W_in = jnp.pad(W_in, ((0, 0), (0, D)))
    W_out = jnp.pad(W_out, ((0, 0), (0, D)))
    pos_s, neg_s = _sc_scores(center, pos, neg_flat, W_in, W_out, B=B, K=K, D=D)
    loss = _tc_loss(pos_s.reshape(B // 128, 128),
                    neg_s.reshape(B * K // 128, 128), B)
    return loss.reshape(())
